# baseline (speedup 1.0000x reference)
import jax
import jax.numpy as jnp
from jax.experimental import pallas as pl
from jax.experimental.pallas import tpu as pltpu

N_GROUPS = 2
CODEBOOK_SIZE = 1024
CODEBOOK_DIM = 128

TOKEN_BLOCK = 2048
CHUNK = 512


def _vq_kernel(x_ref, w_ref, b_ref, cv_ref, out_ref, cw_ref):
    b_row = b_ref[...]
    w = w_ref[...]
    cv = cv_ref[...]
    for c in range(TOKEN_BLOCK // CHUNK):
        r = pl.ds(c * CHUNK, CHUNK)
        cw_parts = []
        for g in range(N_GROUPS):
            x_g = x_ref[r, g * CODEBOOK_DIM:(g + 1) * CODEBOOK_DIM]
            logits = jax.lax.dot_general(
                x_g, w, (((1,), (1,)), ((), ())),
                preferred_element_type=jnp.float32,
            ) + b_row
            m = jnp.max(logits, axis=-1, keepdims=True)
            idx = jax.lax.broadcasted_iota(jnp.int32, logits.shape, 1)
            cw = jnp.min(jnp.where(logits == m, idx, CODEBOOK_SIZE),
                         axis=-1, keepdims=True)
            cw_parts.append(cw)
            e = jnp.exp(logits - m)
            s = jnp.sum(e, axis=-1, keepdims=True)
            acc = jax.lax.dot_general(
                e, cv, (((1,), (0,)), ((), ())),
                preferred_element_type=jnp.float32,
            )
            out_ref[r, g * CODEBOOK_DIM:(g + 1) * CODEBOOK_DIM] = acc / s
        cw_ref[r, :] = jnp.concatenate(cw_parts, axis=1)


def kernel(inputs, attention_mask, W, b, codevectors_table):
    Bb, S, H = inputs.shape
    T = Bb * S
    x = inputs.reshape(T, H)
    b2 = b.reshape(1, CODEBOOK_SIZE)
    grid = (T // TOKEN_BLOCK,)
    out, cw = pl.pallas_call(
        _vq_kernel,
        grid=grid,
        in_specs=[
            pl.BlockSpec((TOKEN_BLOCK, H), lambda i: (i, 0)),
            pl.BlockSpec((CODEBOOK_SIZE, CODEBOOK_DIM), lambda i: (0, 0)),
            pl.BlockSpec((1, CODEBOOK_SIZE), lambda i: (0, 0)),
            pl.BlockSpec((CODEBOOK_SIZE, CODEBOOK_DIM), lambda i: (0, 0)),
        ],
        out_specs=[
            pl.BlockSpec((TOKEN_BLOCK, H), lambda i: (i, 0)),
            pl.BlockSpec((TOKEN_BLOCK, N_GROUPS), lambda i: (i, 0)),
        ],
        out_shape=[
            jax.ShapeDtypeStruct((T, H), jnp.float32),
            jax.ShapeDtypeStruct((T, N_GROUPS), jnp.int32),
        ],
        compiler_params=pltpu.CompilerParams(
            dimension_semantics=("arbitrary",),
        ),
    )(x, W, b2, codevectors_table)
    codevectors = out.reshape(Bb, S, H)
    codewords = cw.reshape(Bb, S, N_GROUPS)
    m = attention_mask[..., None]
    codevectors = jnp.where(m, codevectors, jnp.zeros_like(codevectors))
    codewords = jnp.where(m, codewords, jnp.zeros_like(codewords))
    return codevectors, jax.lax.stop_gradient(codewords)


# speedup vs baseline: 1.2336x; 1.2336x over previous
import jax
import jax.numpy as jnp
from jax.experimental import pallas as pl
from jax.experimental.pallas import tpu as pltpu

N_GROUPS = 2
CODEBOOK_SIZE = 1024
CODEBOOK_DIM = 128

TOKEN_BLOCK = 2048


def _vq_kernel(x_ref, w_ref, b_ref, cv_ref, out_ref, cw_ref):
    b_row = b_ref[...]
    w = w_ref[...]
    cv = cv_ref[...]
    logits_g = []
    for g in range(N_GROUPS):
        x_g = x_ref[:, g * CODEBOOK_DIM:(g + 1) * CODEBOOK_DIM]
        logits_g.append(jax.lax.dot_general(
            x_g, w, (((1,), (1,)), ((), ())),
            preferred_element_type=jnp.float32,
        ) + b_row)
    cw_parts = []
    for g in range(N_GROUPS):
        logits = logits_g[g]
        m = jnp.max(logits, axis=-1, keepdims=True)
        idx = jax.lax.broadcasted_iota(jnp.int32, logits.shape, 1)
        cw = jnp.min(jnp.where(logits == m, idx, CODEBOOK_SIZE),
                     axis=-1, keepdims=True)
        cw_parts.append(cw)
        e = jnp.exp(logits - m)
        s = jnp.sum(e, axis=-1, keepdims=True)
        acc = jax.lax.dot_general(
            e, cv, (((1,), (0,)), ((), ())),
            preferred_element_type=jnp.float32,
        )
        out_ref[:, g * CODEBOOK_DIM:(g + 1) * CODEBOOK_DIM] = acc / s
    cw_ref[...] = jnp.concatenate(cw_parts, axis=1)


def kernel(inputs, attention_mask, W, b, codevectors_table):
    Bb, S, H = inputs.shape
    T = Bb * S
    x = inputs.reshape(T, H)
    b2 = b.reshape(1, CODEBOOK_SIZE)
    grid = (T // TOKEN_BLOCK,)
    out, cw = pl.pallas_call(
        _vq_kernel,
        grid=grid,
        in_specs=[
            pl.BlockSpec((TOKEN_BLOCK, H), lambda i: (i, 0)),
            pl.BlockSpec((CODEBOOK_SIZE, CODEBOOK_DIM), lambda i: (0, 0)),
            pl.BlockSpec((1, CODEBOOK_SIZE), lambda i: (0, 0)),
            pl.BlockSpec((CODEBOOK_SIZE, CODEBOOK_DIM), lambda i: (0, 0)),
        ],
        out_specs=[
            pl.BlockSpec((TOKEN_BLOCK, H), lambda i: (i, 0)),
            pl.BlockSpec((TOKEN_BLOCK, N_GROUPS), lambda i: (i, 0)),
        ],
        out_shape=[
            jax.ShapeDtypeStruct((T, H), jnp.float32),
            jax.ShapeDtypeStruct((T, N_GROUPS), jnp.int32),
        ],
        compiler_params=pltpu.CompilerParams(
            dimension_semantics=("arbitrary",),
        ),
    )(x, W, b2, codevectors_table)
    codevectors = out.reshape(Bb, S, H)
    codewords = cw.reshape(Bb, S, N_GROUPS)
    m = attention_mask[..., None]
    codevectors = jnp.where(m, codevectors, jnp.zeros_like(codevectors))
    codewords = jnp.where(m, codewords, jnp.zeros_like(codewords))
    return codevectors, jax.lax.stop_gradient(codewords)
